# baseline (speedup 1.0000x reference)
import functools

import jax
import jax.numpy as jnp
from jax import lax
from jax.experimental import pallas as pl
from jax.experimental.pallas import tpu as pltpu
from jax.experimental.pallas import tpu_sc as plsc

_NUM_CORES = 2
_NUM_SUBCORES = 16
_NW = _NUM_CORES * _NUM_SUBCORES
_CHUNK = 128


def _sc_gather(table, idx, n_rows, dims):
    rows_per_w = n_rows // _NW
    n_chunks = rows_per_w // _CHUNK
    mesh = plsc.VectorSubcoreMesh(core_axis_name="c", subcore_axis_name="s")

    @functools.partial(
        pl.kernel,
        mesh=mesh,
        out_type=jax.ShapeDtypeStruct((n_rows, dims), jnp.float32),
        scratch_types=[
            pltpu.VMEM((rows_per_w,), jnp.int32),
            pltpu.VMEM((rows_per_w, dims), jnp.float32),
            pltpu.SemaphoreType.DMA,
        ],
        compiler_params=pltpu.CompilerParams(use_tc_tiling_on_sc=False),
    )
    def gather_kernel(table_hbm, idx_hbm, out_hbm, idx_v, rows_v, sem):
        wid = lax.axis_index("s") * _NUM_CORES + lax.axis_index("c")
        base = wid * rows_per_w
        pltpu.sync_copy(idx_hbm.at[pl.ds(base, rows_per_w)], idx_v)
        copies = []
        for c in range(n_chunks):
            copies.append(
                pltpu.async_copy(
                    table_hbm.at[idx_v.at[pl.ds(c * _CHUNK, _CHUNK)]],
                    rows_v.at[pl.ds(c * _CHUNK, _CHUNK)],
                    sem,
                )
            )
        for cp in copies:
            cp.wait()
        pltpu.sync_copy(rows_v, out_hbm.at[pl.ds(base, rows_per_w)])

    return gather_kernel(table, idx)


def _mlp_body(es_ref, ed_ref, w1_ref, b1_ref, w2_ref, b2_ref, w3_ref, b3_ref,
              out_ref):
    e = es_ref[...] + ed_ref[...]
    h = jnp.dot(e, w1_ref[...], preferred_element_type=jnp.float32)
    h = jnp.maximum(h + b1_ref[...], 0.0)
    h = jnp.dot(h, w2_ref[...], preferred_element_type=jnp.float32)
    h = jnp.maximum(h + b2_ref[...], 0.0)
    h = jnp.dot(h, w3_ref[...], preferred_element_type=jnp.float32)
    h = jnp.maximum(h + b3_ref[...], 0.0)
    m = jnp.max(h, axis=1, keepdims=True)
    ex = jnp.exp(h - m)
    out_ref[...] = ex / jnp.sum(ex, axis=1, keepdims=True)


def _tc_mlp(gathered, W1, b1, W2, b2, W3, b3, batch, blk):
    n_cls = W3.shape[1]
    dims = W1.shape[0]
    grid = batch // blk
    full = lambda a: pl.BlockSpec(a.shape, lambda i: (0,) * a.ndim)
    return pl.pallas_call(
        _mlp_body,
        grid=(grid,),
        in_specs=[
            pl.BlockSpec((blk, dims), lambda i: (i, 0)),
            pl.BlockSpec((blk, dims), lambda i: (i + grid, 0)),
            full(W1), full(b1), full(W2), full(b2), full(W3), full(b3),
        ],
        out_specs=pl.BlockSpec((blk, n_cls), lambda i: (i, 0)),
        out_shape=jax.ShapeDtypeStruct((batch, n_cls), jnp.float32),
    )(gathered, gathered, W1, b1, W2, b2, W3, b3)


def kernel(src, dst, table, W1, b1, W2, b2, W3, b3):
    batch = src.shape[0]
    dims = table.shape[1]
    idx = jnp.concatenate([src.astype(jnp.int32), dst.astype(jnp.int32)])
    gathered = _sc_gather(table, idx, 2 * batch, dims)
    return _tc_mlp(
        gathered,
        W1, b1.reshape(1, -1),
        W2, b2.reshape(1, -1),
        W3, b3.reshape(1, -1),
        batch, blk=2048,
    )
